# Initial kernel scaffold; baseline (speedup 1.0000x reference)
import jax
import jax.numpy as jnp
from jax.experimental import pallas as pl


def kernel(x, edge_index0, edge_index1, edge_index2, W_l0, b_l0, W_r0, W_l1, b_l1, W_r1, W_l2, b_l2, W_r2, gamma0, beta0, gamma1, beta1, head1_W, head1_b, head2_W, head2_b):
    raise NotImplementedError("write your pallas kernel here")


# speedup vs baseline: 8.0430x; 8.0430x over previous
import functools

import jax
import jax.numpy as jnp
from jax import lax
from jax.experimental import pallas as pl
from jax.experimental.pallas import tpu as pltpu
from jax.experimental.pallas import tpu_sc as plsc

N = 10000
E = 320000
D = 128
H = 16

NC, NS = 2, 16
NW = NC * NS
EPW = E // NW
CHUNK = 128
NFULL = EPW // CHUNK
TAIL = EPW - NFULL * CHUNK
NP = 10240
RPT = NP // NS

_mesh = plsc.VectorSubcoreMesh(
    core_axis_name="c", subcore_axis_name="s", num_cores=NC, num_subcores=NS)


def _seg_body(y, src, dst, out0, out1, cnt0, cnt1,
              srcb, dstb, rows, srct, dstt, rowst, ones, onest, zb, zb1,
              acc, cacc, sem):
    cid = lax.axis_index("c")
    sid = lax.axis_index("s")
    wid = sid * NC + cid

    z16 = jnp.zeros((16,), jnp.float32)
    o16 = jnp.ones((16,), jnp.float32)
    for i in range(zb.shape[0]):
        zb[i] = z16
    for i in range(RPT // 16):
        zb1[pl.ds(i * 16, 16)] = z16
    for i in range(CHUNK // 16):
        ones[pl.ds(i * 16, 16)] = o16
    onest[...] = o16

    rbase = sid * RPT
    for j in range(RPT // 64):
        pltpu.sync_copy(zb, acc.at[pl.ds(rbase + j * 64, 64)])
    pltpu.sync_copy(zb1, cacc.at[pl.ds(rbase, RPT)])
    plsc.subcore_barrier()

    ebase = wid * EPW

    def body(j, carry):
        cb = ebase + j * CHUNK
        pltpu.sync_copy(src.at[pl.ds(cb, CHUNK)], srcb)
        pltpu.sync_copy(dst.at[pl.ds(cb, CHUNK)], dstb)
        pltpu.async_copy(y.at[srcb], rows, sem).wait()
        pltpu.sync_copy(rows, acc.at[dstb], add=True)
        pltpu.sync_copy(ones, cacc.at[dstb], add=True)
        return carry

    lax.fori_loop(0, NFULL, body, 0)

    tb = ebase + NFULL * CHUNK
    pltpu.sync_copy(src.at[pl.ds(tb, TAIL)], srct)
    pltpu.sync_copy(dst.at[pl.ds(tb, TAIL)], dstt)
    pltpu.async_copy(y.at[srct], rowst, sem).wait()
    pltpu.sync_copy(rowst, acc.at[dstt], add=True)
    pltpu.sync_copy(onest, cacc.at[dstt], add=True)

    plsc.subcore_barrier()

    last = (NS - 1) * RPT

    @pl.when(jnp.logical_and(sid < NS - 1, cid == 0))
    def _():
        pltpu.sync_copy(acc.at[pl.ds(rbase, RPT)], out0.at[pl.ds(rbase, RPT)])
        pltpu.sync_copy(cacc.at[pl.ds(rbase, RPT)], cnt0.at[pl.ds(rbase, RPT)])

    @pl.when(jnp.logical_and(sid == NS - 1, cid == 0))
    def _():
        pltpu.sync_copy(acc.at[pl.ds(last, N - last)], out0.at[pl.ds(last, N - last)])
        pltpu.sync_copy(cacc.at[pl.ds(last, N - last)], cnt0.at[pl.ds(last, N - last)])

    @pl.when(jnp.logical_and(sid < NS - 1, cid == 1))
    def _():
        pltpu.sync_copy(acc.at[pl.ds(rbase, RPT)], out1.at[pl.ds(rbase, RPT)])
        pltpu.sync_copy(cacc.at[pl.ds(rbase, RPT)], cnt1.at[pl.ds(rbase, RPT)])

    @pl.when(jnp.logical_and(sid == NS - 1, cid == 1))
    def _():
        pltpu.sync_copy(acc.at[pl.ds(last, N - last)], out1.at[pl.ds(last, N - last)])
        pltpu.sync_copy(cacc.at[pl.ds(last, N - last)], cnt1.at[pl.ds(last, N - last)])


_seg_call = pl.kernel(
    _seg_body,
    out_type=(
        jax.ShapeDtypeStruct((N, H), jnp.float32),
        jax.ShapeDtypeStruct((N, H), jnp.float32),
        jax.ShapeDtypeStruct((N,), jnp.float32),
        jax.ShapeDtypeStruct((N,), jnp.float32),
    ),
    mesh=_mesh,
    scratch_types=[
        pltpu.VMEM((CHUNK,), jnp.int32),
        pltpu.VMEM((CHUNK,), jnp.int32),
        pltpu.VMEM((CHUNK, H), jnp.float32),
        pltpu.VMEM((TAIL,), jnp.int32),
        pltpu.VMEM((TAIL,), jnp.int32),
        pltpu.VMEM((TAIL, H), jnp.float32),
        pltpu.VMEM((CHUNK,), jnp.float32),
        pltpu.VMEM((TAIL,), jnp.float32),
        pltpu.VMEM((64, H), jnp.float32),
        pltpu.VMEM((RPT,), jnp.float32),
        pltpu.VMEM_SHARED((NP, H), jnp.float32),
        pltpu.VMEM_SHARED((NP,), jnp.float32),
        pltpu.SemaphoreType.DMA,
    ],
    compiler_params=pltpu.CompilerParams(use_tc_tiling_on_sc=False),
)


BS = 1000
G = N // BS


def _proj_body(x_ref, wl_ref, wr_ref, y_ref, z_ref):
    xb = x_ref[...]
    y_ref[...] = jnp.dot(xb, wl_ref[...], preferred_element_type=jnp.float32)
    z_ref[...] = jnp.dot(xb, wr_ref[...], preferred_element_type=jnp.float32)


def _comb_body(p0, p1, c0, c1, z, blp, g, b, wl, wr, y_ref, z_ref):
    m = (p0[...] + p1[...]) / jnp.maximum(c0[...] + c1[...], 1.0)
    h = (m + z[...] + blp[...]) * g[...] + b[...]
    h = jnp.maximum(h, 0.0)
    y_ref[...] = jnp.dot(h, wl[...], preferred_element_type=jnp.float32)
    z_ref[...] = jnp.dot(h, wr[...], preferred_element_type=jnp.float32)


def _lsm(t):
    mx = jnp.max(t, axis=1, keepdims=True)
    e = jnp.exp(t - mx)
    return (t - mx) - jnp.log(jnp.sum(e, axis=1, keepdims=True))


def _fin_body(p0, p1, c0, c1, z, blp, w1, b1, w2, b2, o1, o2, o3):
    h = (p0[...] + p1[...]) / jnp.maximum(c0[...] + c1[...], 1.0) \
        + z[...] + blp[...]
    t1 = jnp.dot(h, w1[...], preferred_element_type=jnp.float32) + b1[...]
    o1[...] = _lsm(t1)
    t2 = jnp.dot(h, w2[...], preferred_element_type=jnp.float32) + b2[...]
    o2[...] = _lsm(t2)
    o3[...] = _lsm(h)


def _row_spec(w):
    return pl.BlockSpec((BS, w), lambda i: (i, 0))


def _full_spec(r, w):
    return pl.BlockSpec((r, w), lambda i: (0, 0))


def kernel(x, edge_index0, edge_index1, edge_index2, W_l0, b_l0, W_r0,
           W_l1, b_l1, W_r1, W_l2, b_l2, W_r2, gamma0, beta0, gamma1, beta1,
           head1_W, head1_b, head2_W, head2_b):
    f32 = jnp.float32
    bn_s = 1.0 / jnp.sqrt(jnp.asarray(1.0 + 1e-5, f32))
    g0 = (gamma0 * bn_s).reshape(1, H)
    g1 = (gamma1 * bn_s).reshape(1, H)
    b0 = beta0.reshape(1, H)
    b1 = beta1.reshape(1, H)

    y0, z0 = pl.pallas_call(
        _proj_body,
        grid=(G,),
        in_specs=[_row_spec(D), _full_spec(D, H), _full_spec(D, H)],
        out_specs=[_row_spec(H), _row_spec(H)],
        out_shape=[jax.ShapeDtypeStruct((N, H), f32)] * 2,
    )(x, W_l0.T, W_r0.T)

    def agg(y, ei):
        p0, p1, c0, c1 = _seg_call(y, ei[0], ei[1])
        return p0, p1, c0.reshape(N, 1), c1.reshape(N, 1)

    def comb(parts, z, blp, g, b, Wl, Wr):
        p0, p1, c0, c1 = parts
        return pl.pallas_call(
            _comb_body,
            grid=(G,),
            in_specs=[_row_spec(H), _row_spec(H), _row_spec(1), _row_spec(1),
                      _row_spec(H), _full_spec(1, H), _full_spec(1, H),
                      _full_spec(1, H), _full_spec(H, H), _full_spec(H, H)],
            out_specs=[_row_spec(H), _row_spec(H)],
            out_shape=[jax.ShapeDtypeStruct((N, H), f32)] * 2,
        )(p0, p1, c0, c1, z, blp.reshape(1, H), g, b, Wl.T, Wr.T)

    parts0 = agg(y0, edge_index0)
    y1, z1 = comb(parts0, z0, b_l0, g0, b0, W_l1, W_r1)
    parts1 = agg(y1, edge_index1)
    y2, z2 = comb(parts1, z1, b_l1, g1, b1, W_l2, W_r2)
    p0, p1, c0, c1 = agg(y2, edge_index2)

    out1, out2, hls = pl.pallas_call(
        _fin_body,
        grid=(G,),
        in_specs=[_row_spec(H), _row_spec(H), _row_spec(1), _row_spec(1),
                  _row_spec(H), _full_spec(1, H),
                  _full_spec(H, 4), _full_spec(1, 4),
                  _full_spec(H, 3), _full_spec(1, 3)],
        out_specs=[_row_spec(4), _row_spec(3), _row_spec(H)],
        out_shape=[jax.ShapeDtypeStruct((N, 4), f32),
                   jax.ShapeDtypeStruct((N, 3), f32),
                   jax.ShapeDtypeStruct((N, H), f32)],
    )(p0, p1, c0, c1, z2, b_l2.reshape(1, H),
      head1_W.T, head1_b.reshape(1, 4), head2_W.T, head2_b.reshape(1, 3))

    return (out1, out2, hls)


# speedup vs baseline: 19.0831x; 2.3726x over previous
import functools

import jax
import jax.numpy as jnp
from jax import lax
from jax.experimental import pallas as pl
from jax.experimental.pallas import tpu as pltpu
from jax.experimental.pallas import tpu_sc as plsc

N = 10000
E = 320000
D = 128
H = 16

NC, NS = 2, 16
NW = NC * NS
EPW = E // NW
CHUNK = 80
NCH = EPW // CHUNK
RB = 5
GRP = NCH // RB
NP = 10240
RPT = NP // NS

_mesh = plsc.VectorSubcoreMesh(
    core_axis_name="c", subcore_axis_name="s", num_cores=NC, num_subcores=NS)


def _seg_body(y, src, dst, out0, out1, cnt0, cnt1,
              srcall, dstall, rows, ones, zb, zb1,
              acc, cacc, gsems, ssems):
    cid = lax.axis_index("c")
    sid = lax.axis_index("s")
    wid = sid * NC + cid

    z16 = jnp.zeros((16,), jnp.float32)
    o16 = jnp.ones((16,), jnp.float32)
    for i in range(zb.shape[0]):
        zb[i] = z16
    for i in range(RPT // 16):
        zb1[pl.ds(i * 16, 16)] = z16
    for i in range(CHUNK // 16):
        ones[pl.ds(i * 16, 16)] = o16

    pltpu.sync_copy(src.at[pl.ds(wid * NCH, NCH)], srcall)
    pltpu.sync_copy(dst.at[pl.ds(wid * NCH, NCH)], dstall)

    rbase = sid * RPT
    for j in range(RPT // 64):
        pltpu.sync_copy(zb, acc.at[pl.ds(rbase + j * 64, 64)])
    pltpu.sync_copy(zb1, cacc.at[pl.ds(rbase, RPT)])
    plsc.subcore_barrier()

    for b in range(RB):
        pltpu.async_copy(y.at[srcall.at[b]], rows.at[b], gsems.at[b])

    def body(g, carry):
        for b in range(RB):
            c = g * RB + b
            pltpu.make_async_copy(y.at[srcall.at[c]], rows.at[b],
                                  gsems.at[b]).wait()
            d1 = pltpu.async_copy(rows.at[b], acc.at[dstall.at[c]],
                                  ssems.at[b], add=True)
            d2 = pltpu.async_copy(ones, cacc.at[dstall.at[c]],
                                  ssems.at[b], add=True)
            d1.wait()
            d2.wait()

            @pl.when(c + RB < NCH)
            def _():
                pltpu.async_copy(y.at[srcall.at[c + RB]], rows.at[b],
                                 gsems.at[b])
        return carry

    lax.fori_loop(0, GRP, body, 0)

    plsc.subcore_barrier()

    last = (NS - 1) * RPT

    @pl.when(jnp.logical_and(sid < NS - 1, cid == 0))
    def _():
        pltpu.sync_copy(acc.at[pl.ds(rbase, RPT)], out0.at[pl.ds(rbase, RPT)])
        pltpu.sync_copy(cacc.at[pl.ds(rbase, RPT)], cnt0.at[pl.ds(rbase, RPT)])

    @pl.when(jnp.logical_and(sid == NS - 1, cid == 0))
    def _():
        pltpu.sync_copy(acc.at[pl.ds(last, N - last)], out0.at[pl.ds(last, N - last)])
        pltpu.sync_copy(cacc.at[pl.ds(last, N - last)], cnt0.at[pl.ds(last, N - last)])

    @pl.when(jnp.logical_and(sid < NS - 1, cid == 1))
    def _():
        pltpu.sync_copy(acc.at[pl.ds(rbase, RPT)], out1.at[pl.ds(rbase, RPT)])
        pltpu.sync_copy(cacc.at[pl.ds(rbase, RPT)], cnt1.at[pl.ds(rbase, RPT)])

    @pl.when(jnp.logical_and(sid == NS - 1, cid == 1))
    def _():
        pltpu.sync_copy(acc.at[pl.ds(last, N - last)], out1.at[pl.ds(last, N - last)])
        pltpu.sync_copy(cacc.at[pl.ds(last, N - last)], cnt1.at[pl.ds(last, N - last)])


_seg_call = pl.kernel(
    _seg_body,
    out_type=(
        jax.ShapeDtypeStruct((N, H), jnp.float32),
        jax.ShapeDtypeStruct((N, H), jnp.float32),
        jax.ShapeDtypeStruct((N,), jnp.float32),
        jax.ShapeDtypeStruct((N,), jnp.float32),
    ),
    mesh=_mesh,
    scratch_types=[
        pltpu.VMEM((NCH, CHUNK), jnp.int32),
        pltpu.VMEM((NCH, CHUNK), jnp.int32),
        pltpu.VMEM((RB, CHUNK, H), jnp.float32),
        pltpu.VMEM((CHUNK,), jnp.float32),
        pltpu.VMEM((64, H), jnp.float32),
        pltpu.VMEM((RPT,), jnp.float32),
        pltpu.VMEM_SHARED((NP, H), jnp.float32),
        pltpu.VMEM_SHARED((NP,), jnp.float32),
        pltpu.SemaphoreType.DMA((RB,)),
        pltpu.SemaphoreType.DMA((RB,)),
    ],
    compiler_params=pltpu.CompilerParams(use_tc_tiling_on_sc=False),
)


BS = 1000
G = N // BS


def _proj_body(x_ref, wl_ref, wr_ref, y_ref, z_ref):
    xb = x_ref[...]
    y_ref[...] = jnp.dot(xb, wl_ref[...], preferred_element_type=jnp.float32)
    z_ref[...] = jnp.dot(xb, wr_ref[...], preferred_element_type=jnp.float32)


def _comb_body(p0, p1, c0, c1, z, blp, g, b, wl, wr, y_ref, z_ref):
    m = (p0[...] + p1[...]) / jnp.maximum(c0[...] + c1[...], 1.0)
    h = (m + z[...] + blp[...]) * g[...] + b[...]
    h = jnp.maximum(h, 0.0)
    y_ref[...] = jnp.dot(h, wl[...], preferred_element_type=jnp.float32)
    z_ref[...] = jnp.dot(h, wr[...], preferred_element_type=jnp.float32)


def _lsm(t):
    mx = jnp.max(t, axis=1, keepdims=True)
    e = jnp.exp(t - mx)
    return (t - mx) - jnp.log(jnp.sum(e, axis=1, keepdims=True))


def _fin_body(p0, p1, c0, c1, z, blp, w1, b1, w2, b2, o1, o2, o3):
    h = (p0[...] + p1[...]) / jnp.maximum(c0[...] + c1[...], 1.0) \
        + z[...] + blp[...]
    t1 = jnp.dot(h, w1[...], preferred_element_type=jnp.float32) + b1[...]
    o1[...] = _lsm(t1)
    t2 = jnp.dot(h, w2[...], preferred_element_type=jnp.float32) + b2[...]
    o2[...] = _lsm(t2)
    o3[...] = _lsm(h)


def _row_spec(w):
    return pl.BlockSpec((BS, w), lambda i: (i, 0))


def _full_spec(r, w):
    return pl.BlockSpec((r, w), lambda i: (0, 0))


def kernel(x, edge_index0, edge_index1, edge_index2, W_l0, b_l0, W_r0,
           W_l1, b_l1, W_r1, W_l2, b_l2, W_r2, gamma0, beta0, gamma1, beta1,
           head1_W, head1_b, head2_W, head2_b):
    f32 = jnp.float32
    bn_s = 1.0 / jnp.sqrt(jnp.asarray(1.0 + 1e-5, f32))
    g0 = (gamma0 * bn_s).reshape(1, H)
    g1 = (gamma1 * bn_s).reshape(1, H)
    b0 = beta0.reshape(1, H)
    b1 = beta1.reshape(1, H)

    y0, z0 = pl.pallas_call(
        _proj_body,
        grid=(G,),
        in_specs=[_row_spec(D), _full_spec(D, H), _full_spec(D, H)],
        out_specs=[_row_spec(H), _row_spec(H)],
        out_shape=[jax.ShapeDtypeStruct((N, H), f32)] * 2,
    )(x, W_l0.T, W_r0.T)

    def agg(y, ei):
        src2 = ei[0].reshape(E // CHUNK, CHUNK)
        dst2 = ei[1].reshape(E // CHUNK, CHUNK)
        p0, p1, c0, c1 = _seg_call(y, src2, dst2)
        return p0, p1, c0.reshape(N, 1), c1.reshape(N, 1)

    def comb(parts, z, blp, g, b, Wl, Wr):
        p0, p1, c0, c1 = parts
        return pl.pallas_call(
            _comb_body,
            grid=(G,),
            in_specs=[_row_spec(H), _row_spec(H), _row_spec(1), _row_spec(1),
                      _row_spec(H), _full_spec(1, H), _full_spec(1, H),
                      _full_spec(1, H), _full_spec(H, H), _full_spec(H, H)],
            out_specs=[_row_spec(H), _row_spec(H)],
            out_shape=[jax.ShapeDtypeStruct((N, H), f32)] * 2,
        )(p0, p1, c0, c1, z, blp.reshape(1, H), g, b, Wl.T, Wr.T)

    parts0 = agg(y0, edge_index0)
    y1, z1 = comb(parts0, z0, b_l0, g0, b0, W_l1, W_r1)
    parts1 = agg(y1, edge_index1)
    y2, z2 = comb(parts1, z1, b_l1, g1, b1, W_l2, W_r2)
    p0, p1, c0, c1 = agg(y2, edge_index2)

    out1, out2, hls = pl.pallas_call(
        _fin_body,
        grid=(G,),
        in_specs=[_row_spec(H), _row_spec(H), _row_spec(1), _row_spec(1),
                  _row_spec(H), _full_spec(1, H),
                  _full_spec(H, 4), _full_spec(1, 4),
                  _full_spec(H, 3), _full_spec(1, 3)],
        out_specs=[_row_spec(4), _row_spec(3), _row_spec(H)],
        out_shape=[jax.ShapeDtypeStruct((N, 4), f32),
                   jax.ShapeDtypeStruct((N, 3), f32),
                   jax.ShapeDtypeStruct((N, H), f32)],
    )(p0, p1, c0, c1, z2, b_l2.reshape(1, H),
      head1_W.T, head1_b.reshape(1, 4), head2_W.T, head2_b.reshape(1, 3))

    return (out1, out2, hls)


# speedup vs baseline: 31.7255x; 1.6625x over previous
import jax
import jax.numpy as jnp
from jax import lax
from jax.experimental import pallas as pl
from jax.experimental.pallas import tpu as pltpu
from jax.experimental.pallas import tpu_sc as plsc

N = 10000
E = 320000
D = 128
H = 16

NC, NS = 2, 16
NW = NC * NS
EPW = E // NW
CHUNK = 80
NCH = EPW // CHUNK
ECH = E // CHUNK
RB = 5
GRP = NCH // RB
NP = 10240
RPT = NP // NS
PR = N * H // 128

_mesh = plsc.VectorSubcoreMesh(
    core_axis_name="c", subcore_axis_name="s", num_cores=NC, num_subcores=NS)


def _seg_body(y, eic, out0, out1, cnt0, cnt1,
              srcall, dstall, rows, ones, zb, zb1, csv, csbf,
              acc, cacc, gsems, ssems):
    cid = lax.axis_index("c")
    sid = lax.axis_index("s")
    wid = sid * NC + cid

    z16 = jnp.zeros((16,), jnp.float32)
    o16 = jnp.ones((16,), jnp.float32)
    for i in range(zb.shape[0]):
        zb[i] = z16
    for i in range(RPT // 16):
        zb1[pl.ds(i * 16, 16)] = z16
    for i in range(CHUNK // 16):
        ones[pl.ds(i * 16, 16)] = o16

    pltpu.sync_copy(eic.at[pl.ds(wid * NCH, NCH)], srcall)
    pltpu.sync_copy(eic.at[pl.ds(ECH + wid * NCH, NCH)], dstall)

    rbase = sid * RPT
    for j in range(RPT // 64):
        pltpu.sync_copy(zb, acc.at[pl.ds(rbase + j * 64, 64)])
    pltpu.sync_copy(zb1, cacc.at[pl.ds(rbase, RPT)])
    plsc.subcore_barrier()

    for b in range(RB):
        pltpu.async_copy(y.at[srcall.at[b]], rows.at[b], gsems.at[b])

    def body(g, carry):
        for b in range(RB):
            c = g * RB + b
            pltpu.make_async_copy(y.at[srcall.at[c]], rows.at[b],
                                  gsems.at[b]).wait()
            d1 = pltpu.async_copy(rows.at[b], acc.at[dstall.at[c]],
                                  ssems.at[b], add=True)
            d2 = pltpu.async_copy(ones, cacc.at[dstall.at[c]],
                                  ssems.at[b], add=True)
            d1.wait()
            d2.wait()

            @pl.when(c + RB < NCH)
            def _():
                pltpu.async_copy(y.at[srcall.at[c + RB]], rows.at[b],
                                 gsems.at[b])
        return carry

    lax.fori_loop(0, GRP, body, 0)

    plsc.subcore_barrier()

    pltpu.sync_copy(cacc.at[pl.ds(rbase, RPT)], csv)

    def sbody(g, carry):
        c16 = csv[pl.ds(g * 16, 16)]
        for k in range(16):
            spl = jnp.take_along_axis(c16, jnp.full((16,), k, jnp.int32),
                                      axis=0)
            csbf[pl.ds(g * 256 + k * 16, 16)] = spl
        return carry

    lax.fori_loop(0, RPT // 16, sbody, 0)

    last = (NS - 1) * RPT
    lastn = N - last

    @pl.when(jnp.logical_and(sid < NS - 1, cid == 0))
    def _():
        pltpu.sync_copy(acc.at[pl.ds(rbase, RPT)], out0.at[pl.ds(rbase, RPT)])
        pltpu.sync_copy(csbf, cnt0.at[pl.ds(rbase * 16, RPT * 16)])

    @pl.when(jnp.logical_and(sid == NS - 1, cid == 0))
    def _():
        pltpu.sync_copy(acc.at[pl.ds(last, lastn)], out0.at[pl.ds(last, lastn)])
        pltpu.sync_copy(csbf.at[pl.ds(0, lastn * 16)],
                        cnt0.at[pl.ds(last * 16, lastn * 16)])

    @pl.when(jnp.logical_and(sid < NS - 1, cid == 1))
    def _():
        pltpu.sync_copy(acc.at[pl.ds(rbase, RPT)], out1.at[pl.ds(rbase, RPT)])
        pltpu.sync_copy(csbf, cnt1.at[pl.ds(rbase * 16, RPT * 16)])

    @pl.when(jnp.logical_and(sid == NS - 1, cid == 1))
    def _():
        pltpu.sync_copy(acc.at[pl.ds(last, lastn)], out1.at[pl.ds(last, lastn)])
        pltpu.sync_copy(csbf.at[pl.ds(0, lastn * 16)],
                        cnt1.at[pl.ds(last * 16, lastn * 16)])


_seg_call = pl.kernel(
    _seg_body,
    out_type=(
        jax.ShapeDtypeStruct((N, H), jnp.float32),
        jax.ShapeDtypeStruct((N, H), jnp.float32),
        jax.ShapeDtypeStruct((N * H,), jnp.float32),
        jax.ShapeDtypeStruct((N * H,), jnp.float32),
    ),
    mesh=_mesh,
    scratch_types=[
        pltpu.VMEM((NCH, CHUNK), jnp.int32),
        pltpu.VMEM((NCH, CHUNK), jnp.int32),
        pltpu.VMEM((RB, CHUNK, H), jnp.float32),
        pltpu.VMEM((CHUNK,), jnp.float32),
        pltpu.VMEM((64, H), jnp.float32),
        pltpu.VMEM((RPT,), jnp.float32),
        pltpu.VMEM((RPT,), jnp.float32),
        pltpu.VMEM((RPT * 16,), jnp.float32),
        pltpu.VMEM_SHARED((NP, H), jnp.float32),
        pltpu.VMEM_SHARED((NP,), jnp.float32),
        pltpu.SemaphoreType.DMA((RB,)),
        pltpu.SemaphoreType.DMA((RB,)),
    ],
    compiler_params=pltpu.CompilerParams(use_tc_tiling_on_sc=False),
)


def _proj_body(x_ref, wl_ref, wr_ref, y_ref, z_ref):
    xb = x_ref[...]
    y_ref[...] = jnp.dot(xb, wl_ref[...], preferred_element_type=jnp.float32)
    z_ref[...] = jnp.dot(xb, wr_ref[...], preferred_element_type=jnp.float32)


def _comb_body(p0, p1, c0, c1, z, blp, g, b, wl, wr, y_ref, z_ref):
    m = (p0[...] + p1[...]) / jnp.maximum(c0[...] + c1[...], 1.0)
    h = (m + z[...] + blp[...]) * g[...] + b[...]
    h = jnp.maximum(h, 0.0)
    y_ref[...] = jnp.dot(h, wl[...], preferred_element_type=jnp.float32)
    z_ref[...] = jnp.dot(h, wr[...], preferred_element_type=jnp.float32)


def _lsm_packed(t, gmat):
    mx = jnp.max(t, axis=1, keepdims=True)
    e = jnp.exp(t - mx)
    s = jnp.dot(e, gmat, preferred_element_type=jnp.float32)
    return (t - mx) - jnp.log(s)


def _fin_body(p0, p1, c0, c1, z, blp, w1, b1, w2, b2, g4, g3, g16,
              o1, o2, o3):
    h = (p0[...] + p1[...]) / jnp.maximum(c0[...] + c1[...], 1.0) \
        + z[...] + blp[...]
    t1 = jnp.dot(h, w1[...], preferred_element_type=jnp.float32) + b1[...]
    o1[...] = _lsm_packed(t1, g4[...])
    t2 = jnp.dot(h, w2[...], preferred_element_type=jnp.float32) + b2[...]
    o2[...] = _lsm_packed(t2, g3[...])
    o3[...] = _lsm_packed(h, g16[...])


def kernel(x, edge_index0, edge_index1, edge_index2, W_l0, b_l0, W_r0,
           W_l1, b_l1, W_r1, W_l2, b_l2, W_r2, gamma0, beta0, gamma1, beta1,
           head1_W, head1_b, head2_W, head2_b):
    f32 = jnp.float32
    eye8 = jnp.eye(8, dtype=f32)
    bn_s = 1.0 / jnp.sqrt(jnp.asarray(1.0 + 1e-5, f32))

    def tile8(v):
        return jnp.tile(v, 8).reshape(1, -1)

    sds = jax.ShapeDtypeStruct

    y0, z0 = pl.pallas_call(
        _proj_body,
        out_shape=[sds((N, H), f32)] * 2,
    )(x, W_l0.T, W_r0.T)
    y0p = y0.reshape(PR, 128)
    z0p = z0.reshape(PR, 128)

    def agg(yp, ei):
        p0, p1, c0f, c1f = _seg_call(yp.reshape(N, H), ei.reshape(2 * ECH, CHUNK))
        return (p0.reshape(PR, 128), p1.reshape(PR, 128),
                c0f.reshape(PR, 128), c1f.reshape(PR, 128))

    def comb(parts, zp, blp, gam, bet, Wl, Wr):
        p0, p1, c0, c1 = parts
        return pl.pallas_call(
            _comb_body,
            out_shape=[sds((PR, 128), f32)] * 2,
        )(p0, p1, c0, c1, zp, tile8(blp), tile8(gam * bn_s), tile8(bet),
          jnp.kron(eye8, Wl.T), jnp.kron(eye8, Wr.T))

    parts0 = agg(y0p, edge_index0)
    y1p, z1p = comb(parts0, z0p, b_l0, gamma0, beta0, W_l1, W_r1)
    parts1 = agg(y1p, edge_index1)
    y2p, z2p = comb(parts1, z1p, b_l1, gamma1, beta1, W_l2, W_r2)
    p0, p1, c0, c1 = agg(y2p, edge_index2)

    o1p, o2p, hlsp = pl.pallas_call(
        _fin_body,
        out_shape=[sds((PR, 32), f32), sds((PR, 24), f32),
                   sds((PR, 128), f32)],
    )(p0, p1, c0, c1, z2p, tile8(b_l2),
      jnp.kron(eye8, head1_W.T), tile8(head1_b),
      jnp.kron(eye8, head2_W.T), tile8(head2_b),
      jnp.kron(eye8, jnp.ones((4, 4), f32)),
      jnp.kron(eye8, jnp.ones((3, 3), f32)),
      jnp.kron(eye8, jnp.ones((16, 16), f32)))

    return (o1p.reshape(N, 4), o2p.reshape(N, 3), hlsp.reshape(N, H))
